# Initial kernel scaffold; baseline (speedup 1.0000x reference)
import jax
import jax.numpy as jnp
from jax.experimental import pallas as pl


def kernel(heads, relations, tails, entity_embedding, relation_embedding):
    raise NotImplementedError("write your pallas kernel here")


# speedup vs baseline: 1.3042x; 1.3042x over previous
import functools

import jax
import jax.numpy as jnp
import numpy as np
from jax import lax
from jax.experimental import pallas as pl
from jax.experimental.pallas import tpu as pltpu
from jax.experimental.pallas import tpu_sc as plsc

_MARGIN = 6.0
_EPSILON = 2.0
_DIM = 128
_EMB_RANGE = (_MARGIN + _EPSILON) / _DIM
_BATCH = 16384
_ENT_D = 2 * _DIM

_NC = 2
_NS = 16
_NW = _NC * _NS

_CHUNK = 128


def _sc_gather_kernel(ent_hbm, rel_hbm, iht_hbm, ir_hbm, out_ht, out_rel,
                      idx_v, buf_ht, buf_rel, sem):
    wid = lax.axis_index("s") * _NC + lax.axis_index("c")
    n_ht = (2 * _BATCH) // _NW
    n_r = _BATCH // _NW
    base_ht = wid * n_ht
    base_r = wid * n_r
    for c in range(n_ht // _CHUNK):
        off = base_ht + c * _CHUNK
        pltpu.sync_copy(iht_hbm.at[pl.ds(off, _CHUNK)], idx_v)
        pltpu.async_copy(ent_hbm.at[idx_v], buf_ht, sem).wait()
        pltpu.sync_copy(buf_ht, out_ht.at[pl.ds(off, _CHUNK)])
    for c in range(n_r // _CHUNK):
        off = base_r + c * _CHUNK
        pltpu.sync_copy(ir_hbm.at[pl.ds(off, _CHUNK)], idx_v)
        pltpu.async_copy(rel_hbm.at[idx_v], buf_rel, sem).wait()
        pltpu.sync_copy(buf_rel, out_rel.at[pl.ds(off, _CHUNK)])


def _sc_gather(entity_embedding, relation_embedding, idx_ht, idx_r):
    mesh = plsc.VectorSubcoreMesh(core_axis_name="c", subcore_axis_name="s")
    run = pl.kernel(
        _sc_gather_kernel,
        out_type=(
            jax.ShapeDtypeStruct((2 * _BATCH, _ENT_D), jnp.float32),
            jax.ShapeDtypeStruct((_BATCH, _DIM), jnp.float32),
        ),
        mesh=mesh,
        scratch_types=[
            pltpu.VMEM((_CHUNK,), jnp.int32),
            pltpu.VMEM((_CHUNK, _ENT_D), jnp.float32),
            pltpu.VMEM((_CHUNK, _DIM), jnp.float32),
            pltpu.SemaphoreType.DMA,
        ],
    )
    return run(entity_embedding, relation_embedding, idx_ht, idx_r)


_BB = 2048


def _tc_score_kernel(h_ref, t_ref, r_ref, o_ref):
    re_h = h_ref[:, :_DIM]
    im_h = h_ref[:, _DIM:]
    re_t = t_ref[:, :_DIM]
    im_t = t_ref[:, _DIM:]
    phase = r_ref[...] * np.float32(np.pi / _EMB_RANGE)
    re_r = jnp.cos(phase)
    im_r = jnp.sin(phase)
    re_rot = re_h * re_r - im_h * im_r
    im_rot = re_h * im_r + im_h * re_r
    d_re = re_rot - re_t
    d_im = im_rot - im_t
    score = d_re * d_re + d_im * d_im
    acc = jnp.sum(score * score, axis=1)
    o_ref[...] = _MARGIN - jnp.sqrt(acc)


def _tc_score(ht, relg):
    nblk = _BATCH // _BB
    return pl.pallas_call(
        _tc_score_kernel,
        grid=(nblk,),
        in_specs=[
            pl.BlockSpec((_BB, _ENT_D), lambda i: (i, 0)),
            pl.BlockSpec((_BB, _ENT_D), lambda i: (i + nblk, 0)),
            pl.BlockSpec((_BB, _DIM), lambda i: (i, 0)),
        ],
        out_specs=pl.BlockSpec((_BB,), lambda i: (i,)),
        out_shape=jax.ShapeDtypeStruct((_BATCH,), jnp.float32),
        compiler_params=pltpu.CompilerParams(
            dimension_semantics=("parallel",)),
    )(ht, ht, relg)


@jax.jit
def kernel(heads, relations, tails, entity_embedding, relation_embedding):
    heads = heads.astype(jnp.int32)
    tails = tails.astype(jnp.int32)
    relations = relations.astype(jnp.int32)
    idx_ht = jnp.concatenate([heads, tails])
    ht, relg = _sc_gather(entity_embedding, relation_embedding, idx_ht,
                          relations)
    return _tc_score(ht, relg)


# speedup vs baseline: 1.4063x; 1.0783x over previous
import functools

import jax
import jax.numpy as jnp
import numpy as np
from jax import lax
from jax.experimental import pallas as pl
from jax.experimental.pallas import tpu as pltpu
from jax.experimental.pallas import tpu_sc as plsc

_MARGIN = 6.0
_EPSILON = 2.0
_DIM = 128
_EMB_RANGE = (_MARGIN + _EPSILON) / _DIM
_BATCH = 16384
_ENT_D = 2 * _DIM

_NC = 2
_NS = 16
_NW = _NC * _NS

_CHUNK = 128


def _pipelined_gather(table_hbm, idx_v, out_hbm, base, n_rows, bufs, gsem,
                      wsem):
    n = n_rows // _CHUNK
    gh = [None] * n
    wh = [None] * n
    for c in range(n):
        if c >= 1:
            gh[c - 1].wait()
            wh[c - 1] = pltpu.async_copy(
                bufs[(c - 1) % 2],
                out_hbm.at[pl.ds(base + (c - 1) * _CHUNK, _CHUNK)], wsem)
        if c >= 2:
            wh[c - 2].wait()
        gh[c] = pltpu.async_copy(
            table_hbm.at[idx_v.at[pl.ds(c * _CHUNK, _CHUNK)]],
            bufs[c % 2], gsem)
    gh[n - 1].wait()
    wh[n - 1] = pltpu.async_copy(
        bufs[(n - 1) % 2],
        out_hbm.at[pl.ds(base + (n - 1) * _CHUNK, _CHUNK)], wsem)
    if n >= 2:
        wh[n - 2].wait()
    wh[n - 1].wait()


def _sc_gather_kernel(ent_hbm, rel_hbm, iht_hbm, ir_hbm, out_ht, out_rel,
                      iht_v, ir_v, bh0, bh1, br0, br1, gsem, wsem):
    wid = lax.axis_index("s") * _NC + lax.axis_index("c")
    n_ht = (2 * _BATCH) // _NW
    n_r = _BATCH // _NW
    base_ht = wid * n_ht
    base_r = wid * n_r
    pltpu.sync_copy(iht_hbm.at[pl.ds(base_ht, n_ht)], iht_v)
    pltpu.sync_copy(ir_hbm.at[pl.ds(base_r, n_r)], ir_v)
    _pipelined_gather(ent_hbm, iht_v, out_ht, base_ht, n_ht, (bh0, bh1),
                      gsem, wsem)
    _pipelined_gather(rel_hbm, ir_v, out_rel, base_r, n_r, (br0, br1),
                      gsem, wsem)


def _sc_gather(entity_embedding, relation_embedding, idx_ht, idx_r):
    mesh = plsc.VectorSubcoreMesh(core_axis_name="c", subcore_axis_name="s")
    run = pl.kernel(
        _sc_gather_kernel,
        out_type=(
            jax.ShapeDtypeStruct((2 * _BATCH, _ENT_D), jnp.float32),
            jax.ShapeDtypeStruct((_BATCH, _DIM), jnp.float32),
        ),
        mesh=mesh,
        scratch_types=[
            pltpu.VMEM(((2 * _BATCH) // _NW,), jnp.int32),
            pltpu.VMEM((_BATCH // _NW,), jnp.int32),
            pltpu.VMEM((_CHUNK, _ENT_D), jnp.float32),
            pltpu.VMEM((_CHUNK, _ENT_D), jnp.float32),
            pltpu.VMEM((_CHUNK, _DIM), jnp.float32),
            pltpu.VMEM((_CHUNK, _DIM), jnp.float32),
            pltpu.SemaphoreType.DMA,
            pltpu.SemaphoreType.DMA,
        ],
    )
    return run(entity_embedding, relation_embedding, idx_ht, idx_r)


_BB = 2048


def _tc_score_kernel(h_ref, t_ref, r_ref, o_ref):
    re_h = h_ref[:, :_DIM]
    im_h = h_ref[:, _DIM:]
    re_t = t_ref[:, :_DIM]
    im_t = t_ref[:, _DIM:]
    phase = r_ref[...] * np.float32(np.pi / _EMB_RANGE)
    re_r = jnp.cos(phase)
    im_r = jnp.sin(phase)
    re_rot = re_h * re_r - im_h * im_r
    im_rot = re_h * im_r + im_h * re_r
    d_re = re_rot - re_t
    d_im = im_rot - im_t
    score = d_re * d_re + d_im * d_im
    acc = jnp.sum(score * score, axis=1)
    o_ref[...] = _MARGIN - jnp.sqrt(acc)


def _tc_score(ht, relg):
    nblk = _BATCH // _BB
    return pl.pallas_call(
        _tc_score_kernel,
        grid=(nblk,),
        in_specs=[
            pl.BlockSpec((_BB, _ENT_D), lambda i: (i, 0)),
            pl.BlockSpec((_BB, _ENT_D), lambda i: (i + nblk, 0)),
            pl.BlockSpec((_BB, _DIM), lambda i: (i, 0)),
        ],
        out_specs=pl.BlockSpec((_BB,), lambda i: (i,)),
        out_shape=jax.ShapeDtypeStruct((_BATCH,), jnp.float32),
        compiler_params=pltpu.CompilerParams(
            dimension_semantics=("parallel",)),
    )(ht, ht, relg)


@jax.jit
def kernel(heads, relations, tails, entity_embedding, relation_embedding):
    heads = heads.astype(jnp.int32)
    tails = tails.astype(jnp.int32)
    relations = relations.astype(jnp.int32)
    idx_ht = jnp.concatenate([heads, tails])
    ht, relg = _sc_gather(entity_embedding, relation_embedding, idx_ht,
                          relations)
    return _tc_score(ht, relg)


# speedup vs baseline: 1.7878x; 1.2713x over previous
import functools

import jax
import jax.numpy as jnp
import numpy as np
from jax import lax
from jax.experimental import pallas as pl
from jax.experimental.pallas import tpu as pltpu
from jax.experimental.pallas import tpu_sc as plsc

_MARGIN = 6.0
_EPSILON = 2.0
_DIM = 128
_EMB_RANGE = (_MARGIN + _EPSILON) / _DIM
_BATCH = 16384
_ENT_D = 2 * _DIM

_NC = 2
_NS = 16
_NW = _NC * _NS

_CHUNK = 128


def _pipelined_gather(table_hbm, idx_v, out_hbm, base, n_rows, bufs, gsem,
                      wsem):
    n = n_rows // _CHUNK
    gh = [None] * n
    wh = [None] * n
    for c in range(n):
        if c >= 1:
            gh[c - 1].wait()
            wh[c - 1] = pltpu.async_copy(
                bufs[(c - 1) % 2],
                out_hbm.at[pl.ds(base + (c - 1) * _CHUNK, _CHUNK)], wsem)
        if c >= 2:
            wh[c - 2].wait()
        gh[c] = pltpu.async_copy(
            table_hbm.at[idx_v.at[pl.ds(c * _CHUNK, _CHUNK)]],
            bufs[c % 2], gsem)
    gh[n - 1].wait()
    wh[n - 1] = pltpu.async_copy(
        bufs[(n - 1) % 2],
        out_hbm.at[pl.ds(base + (n - 1) * _CHUNK, _CHUNK)], wsem)
    if n >= 2:
        wh[n - 2].wait()
    wh[n - 1].wait()


def _sc_gather_kernel(ent_hbm, rel_hbm, iht_hbm, ir_hbm, out_ht, out_rel,
                      iht_v, ir_v, bh0, bh1, br0, br1, gsem, wsem):
    wid = lax.axis_index("s") * _NC + lax.axis_index("c")
    n_ht = (2 * _BATCH) // _NW
    n_r = _BATCH // _NW
    base_ht = wid * n_ht
    base_r = wid * n_r
    pltpu.sync_copy(iht_hbm.at[pl.ds(base_ht, n_ht)], iht_v)
    pltpu.sync_copy(ir_hbm.at[pl.ds(base_r, n_r)], ir_v)
    _pipelined_gather(ent_hbm, iht_v, out_ht, base_ht, n_ht, (bh0, bh1),
                      gsem, wsem)
    _pipelined_gather(rel_hbm, ir_v, out_rel, base_r, n_r, (br0, br1),
                      gsem, wsem)


def _sc_gather(entity_embedding, relation_embedding, idx_ht, idx_r):
    mesh = plsc.VectorSubcoreMesh(core_axis_name="c", subcore_axis_name="s")
    run = pl.kernel(
        _sc_gather_kernel,
        out_type=(
            jax.ShapeDtypeStruct((2 * _BATCH, _ENT_D), jnp.float32),
            jax.ShapeDtypeStruct((_BATCH, _DIM), jnp.float32),
        ),
        mesh=mesh,
        scratch_types=[
            pltpu.VMEM(((2 * _BATCH) // _NW,), jnp.int32),
            pltpu.VMEM((_BATCH // _NW,), jnp.int32),
            pltpu.VMEM((_CHUNK, _ENT_D), jnp.float32),
            pltpu.VMEM((_CHUNK, _ENT_D), jnp.float32),
            pltpu.VMEM((_CHUNK, _DIM), jnp.float32),
            pltpu.VMEM((_CHUNK, _DIM), jnp.float32),
            pltpu.SemaphoreType.DMA,
            pltpu.SemaphoreType.DMA,
        ],
    )
    return run(entity_embedding, relation_embedding, idx_ht, idx_r)


_BB = 2048


_SIN_C = (9.99999600e-01, -1.66665526e-01, 8.33240285e-03, -1.98086298e-04,
          2.69971060e-06, -2.03620814e-08)
_COS_C = (9.99999989e-01, -4.99999891e-01, 4.16664892e-02, -1.38878034e-03,
          2.47698803e-05, -2.70789985e-07, 1.72449738e-09)


def _poly_sin(x, t):
    acc = jnp.float32(_SIN_C[-1])
    for c in _SIN_C[-2::-1]:
        acc = acc * t + jnp.float32(c)
    return x * acc


def _poly_cos(t):
    acc = jnp.float32(_COS_C[-1])
    for c in _COS_C[-2::-1]:
        acc = acc * t + jnp.float32(c)
    return acc


def _tc_score_kernel(h_ref, t_ref, r_ref, o_ref):
    re_h = h_ref[:, :_DIM]
    im_h = h_ref[:, _DIM:]
    re_t = t_ref[:, :_DIM]
    im_t = t_ref[:, _DIM:]
    phase = r_ref[...] * np.float32(np.pi / _EMB_RANGE)
    t2 = phase * phase
    re_r = _poly_cos(t2)
    im_r = _poly_sin(phase, t2)
    re_rot = re_h * re_r - im_h * im_r
    im_rot = re_h * im_r + im_h * re_r
    d_re = re_rot - re_t
    d_im = im_rot - im_t
    score = d_re * d_re + d_im * d_im
    acc = jnp.sum(score * score, axis=1)
    o_ref[...] = _MARGIN - jnp.sqrt(acc)


def _tc_score(ht, relg):
    nblk = _BATCH // _BB
    return pl.pallas_call(
        _tc_score_kernel,
        grid=(nblk,),
        in_specs=[
            pl.BlockSpec((_BB, _ENT_D), lambda i: (i, 0)),
            pl.BlockSpec((_BB, _ENT_D), lambda i: (i + nblk, 0)),
            pl.BlockSpec((_BB, _DIM), lambda i: (i, 0)),
        ],
        out_specs=pl.BlockSpec((_BB,), lambda i: (i,)),
        out_shape=jax.ShapeDtypeStruct((_BATCH,), jnp.float32),
        compiler_params=pltpu.CompilerParams(
            dimension_semantics=("parallel",)),
    )(ht, ht, relg)


@jax.jit
def kernel(heads, relations, tails, entity_embedding, relation_embedding):
    heads = heads.astype(jnp.int32)
    tails = tails.astype(jnp.int32)
    relations = relations.astype(jnp.int32)
    idx_ht = jnp.concatenate([heads, tails])
    ht, relg = _sc_gather(entity_embedding, relation_embedding, idx_ht,
                          relations)
    return _tc_score(ht, relg)
